# Initial kernel scaffold; baseline (speedup 1.0000x reference)
import jax
import jax.numpy as jnp
from jax.experimental import pallas as pl


def kernel(x, adj, W0, p0, W1, p1, W2, p2, fc1_w, fc1_b, fc2_w, fc2_b):
    raise NotImplementedError("write your pallas kernel here")


# speedup vs baseline: 4.4097x; 4.4097x over previous
import functools

import jax
import jax.numpy as jnp
from jax.experimental import pallas as pl
from jax.experimental.pallas import tpu as pltpu

N = 1024
NEG = -1.0e30
HI = jax.lax.Precision.HIGHEST


def _ax_split(Ab, xm):
    x1 = xm.astype(jnp.bfloat16)
    r1 = xm - x1.astype(jnp.float32)
    x2 = r1.astype(jnp.bfloat16)
    x3 = (r1 - x2.astype(jnp.float32)).astype(jnp.bfloat16)
    return (jnp.dot(Ab, x1, preferred_element_type=jnp.float32)
            + jnp.dot(Ab, x2, preferred_element_type=jnp.float32)
            + jnp.dot(Ab, x3, preferred_element_type=jnp.float32))


def _graph_unet_kernel(ks, x_ref, adj_ref, W0_ref, p0_ref, W1_ref, p1_ref,
                       W2_ref, p2_ref, fc1w_ref, fc1b_ref, fc2w_ref,
                       fc2b_ref, out_ref):
    Ab = adj_ref[0].astype(jnp.bfloat16)
    xm = x_ref[0]

    ia_col = jax.lax.broadcasted_iota(jnp.int32, (N, 1), 0)
    ib_row = jax.lax.broadcasted_iota(jnp.int32, (1, N), 1)
    eye = (jax.lax.broadcasted_iota(jnp.int32, (N, N), 0)
           == jax.lax.broadcasted_iota(jnp.int32, (N, N), 1)
           ).astype(jnp.float32)

    act_col = jnp.ones((N, 1), dtype=jnp.float32)
    act_row = jnp.ones((1, N), dtype=jnp.float32)

    Ws = [W0_ref, W1_ref, W2_ref]
    ps = [p0_ref, p1_ref, p2_ref]
    for i in range(3):
        y = _ax_split(Ab, xm)
        h = jnp.maximum(
            jnp.dot(y, Ws[i][...], preferred_element_type=jnp.float32,
                    precision=HI), 0.0)
        p = ps[i][...]
        pnorm = jnp.sqrt(jnp.sum(p * p)) + 1e-8
        s_col = jax.lax.dot_general(
            h, p, (((1,), (1,)), ((), ())),
            preferred_element_type=jnp.float32, precision=HI) / pnorm
        s_row = jax.lax.dot_general(
            s_col, eye, (((0,), (0,)), ((), ())),
            preferred_element_type=jnp.float32, precision=HI)

        sm_col = jnp.where(act_col > 0.0, s_col, NEG)
        sm_row = jnp.where(act_row > 0.0, s_row, NEG)
        C = (sm_row > sm_col) | ((sm_row == sm_col) & (ib_row < ia_col))
        rank_col = jnp.sum(C.astype(jnp.float32), axis=1, keepdims=True)
        act_col = (rank_col < float(ks[i])).astype(jnp.float32)
        act_row = jax.lax.dot_general(
            act_col, eye, (((0,), (0,)), ((), ())),
            preferred_element_type=jnp.float32, precision=HI)

        xm = h * (act_col * jax.nn.sigmoid(s_col))

    g = jnp.sum(xm, axis=0, keepdims=True) / float(ks[2])
    z = jnp.maximum(
        jnp.dot(g, fc1w_ref[...], preferred_element_type=jnp.float32,
                precision=HI)
        + fc1b_ref[...], 0.0)
    logits = (jnp.dot(z, fc2w_ref[...], preferred_element_type=jnp.float32,
                      precision=HI)
              + fc2b_ref[...])
    m = jnp.max(logits, axis=-1, keepdims=True)
    e = jnp.exp(logits - m)
    out_ref[0] = e / jnp.sum(e, axis=-1, keepdims=True)


def kernel(x, adj, W0, p0, W1, p1, W2, p2, fc1_w, fc1_b, fc2_w, fc2_b):
    B, n, F = x.shape
    C = fc2_w.shape[1]
    ks = []
    kk = n
    for r in (0.8, 0.7, 0.6):
        kk = max(2, int(r * kk))
        ks.append(kk)

    full = lambda shape: pl.BlockSpec(shape, lambda b: (0,) * len(shape))
    grid_spec = pl.GridSpec(
        grid=(B,),
        in_specs=[
            pl.BlockSpec((1, n, F), lambda b: (b, 0, 0)),
            pl.BlockSpec((1, n, n), lambda b: (b, 0, 0)),
            full(W0.shape), full((1, p0.shape[0])),
            full(W1.shape), full((1, p1.shape[0])),
            full(W2.shape), full((1, p2.shape[0])),
            full(fc1_w.shape), full((1, fc1_b.shape[0])),
            full(fc2_w.shape), full((1, fc2_b.shape[0])),
        ],
        out_specs=pl.BlockSpec((1, 1, C), lambda b: (b, 0, 0)),
    )
    out = pl.pallas_call(
        functools.partial(_graph_unet_kernel, tuple(ks)),
        grid_spec=grid_spec,
        out_shape=jax.ShapeDtypeStruct((B, 1, C), jnp.float32),
        compiler_params=pltpu.CompilerParams(
            dimension_semantics=("parallel",)),
    )(x, adj, W0, p0.reshape(1, -1), W1, p1.reshape(1, -1), W2,
      p2.reshape(1, -1), fc1_w, fc1_b.reshape(1, -1), fc2_w,
      fc2_b.reshape(1, -1))
    return out.reshape(B, C)
